# Initial kernel scaffold; baseline (speedup 1.0000x reference)
import jax
import jax.numpy as jnp
from jax.experimental import pallas as pl


def kernel(x, table):
    raise NotImplementedError("write your pallas kernel here")


# speedup vs baseline: 6.0616x; 6.0616x over previous
import functools

import jax
import jax.numpy as jnp
from jax import lax
from jax.experimental import pallas as pl
from jax.experimental.pallas import tpu as pltpu, tpu_sc as plsc

NTOKEN = 100000
DIM = 64
BATCH = 16384
HIST = 50

NC = 2
NS = 16
NW = NC * NS

B_TOTAL = BATCH * HIST
B_PER_W = B_TOTAL // NW

CHUNK = 1024
SUB = 128
K = CHUNK // SUB
NCHUNK = B_PER_W // CHUNK


def _emb_body(x_hbm, table_hbm, out_hbm, idx_v, rows_v, gsem):
  wid = lax.axis_index("s") * NC + lax.axis_index("c")
  base = wid * B_PER_W

  def chunk_body(i, carry):
    off = base + i * CHUNK
    pltpu.sync_copy(x_hbm.at[pl.ds(off, CHUNK)], idx_v)
    copies = []
    for j in range(K):
      copies.append(
          pltpu.async_copy(
              table_hbm.at[idx_v.at[pl.ds(j * SUB, SUB)]],
              rows_v.at[pl.ds(j * SUB, SUB)],
              gsem,
          )
      )
    for c in copies:
      c.wait()
    pltpu.sync_copy(rows_v, out_hbm.at[pl.ds(off, CHUNK)])
    return carry

  lax.fori_loop(0, NCHUNK, chunk_body, 0)


@jax.jit
def _emb(x_flat, table):
  mesh = plsc.VectorSubcoreMesh(core_axis_name="c", subcore_axis_name="s")
  f = pl.kernel(
      _emb_body,
      out_type=jax.ShapeDtypeStruct((B_TOTAL, DIM), jnp.float32),
      mesh=mesh,
      scratch_types=[
          pltpu.VMEM((CHUNK,), jnp.int32),
          pltpu.VMEM((CHUNK, DIM), jnp.float32),
          pltpu.SemaphoreType.DMA,
      ],
      compiler_params=pltpu.CompilerParams(use_tc_tiling_on_sc=False),
  )
  return f(x_flat, table)


def kernel(x, table):
  x_flat = x.reshape(-1).astype(jnp.int32)
  out = _emb(x_flat, table)
  return out.reshape(BATCH, HIST, DIM)


# speedup vs baseline: 6.2220x; 1.0264x over previous
import functools

import jax
import jax.numpy as jnp
from jax import lax
from jax.experimental import pallas as pl
from jax.experimental.pallas import tpu as pltpu, tpu_sc as plsc

NTOKEN = 100000
DIM = 64
BATCH = 16384
HIST = 50

NC = 2
NS = 16
NW = NC * NS

B_TOTAL = BATCH * HIST
B_PER_W = B_TOTAL // NW

CHUNK = 640
SUB = 128
K = CHUNK // SUB
NCHUNK = B_PER_W // CHUNK
NBUF = 2


def _emb_body(x_hbm, table_hbm, out_hbm,
              idx0, idx1, rows0, rows1,
              isem0, isem1, gsem0, gsem1, osem0, osem1):
  idx_v = (idx0, idx1)
  rows_v = (rows0, rows1)
  isem = (isem0, isem1)
  gsem = (gsem0, gsem1)
  osem = (osem0, osem1)

  wid = lax.axis_index("s") * NC + lax.axis_index("c")
  base = wid * B_PER_W

  pltpu.async_copy(x_hbm.at[pl.ds(base, CHUNK)], idx_v[0], isem[0])

  def step(g, carry):
    for b in range(NBUF):
      i = g * NBUF + b
      off = base + i * CHUNK
      nb = (b + 1) % NBUF

      @pl.when(i + 1 < NCHUNK)
      def _():
        pltpu.async_copy(x_hbm.at[pl.ds(off + CHUNK, CHUNK)], idx_v[nb],
                         isem[nb])

      pltpu.make_async_copy(x_hbm.at[pl.ds(off, CHUNK)], idx_v[b],
                            isem[b]).wait()

      @pl.when(i >= NBUF)
      def _():
        pltpu.make_async_copy(rows_v[b],
                              out_hbm.at[pl.ds(off - NBUF * CHUNK, CHUNK)],
                              osem[b]).wait()

      copies = [
          pltpu.async_copy(
              table_hbm.at[idx_v[b].at[pl.ds(j * SUB, SUB)]],
              rows_v[b].at[pl.ds(j * SUB, SUB)],
              gsem[b],
          )
          for j in range(K)
      ]
      for c in copies:
        c.wait()

      pltpu.async_copy(rows_v[b], out_hbm.at[pl.ds(off, CHUNK)], osem[b])
    return carry

  lax.fori_loop(0, NCHUNK // NBUF, step, 0)

  for b in range(NBUF):
    i = NCHUNK - NBUF + b
    pltpu.make_async_copy(rows_v[b],
                          out_hbm.at[pl.ds(base + i * CHUNK, CHUNK)],
                          osem[b]).wait()


@jax.jit
def _emb(x_flat, table):
  mesh = plsc.VectorSubcoreMesh(core_axis_name="c", subcore_axis_name="s")
  f = pl.kernel(
      _emb_body,
      out_type=jax.ShapeDtypeStruct((B_TOTAL, DIM), jnp.float32),
      mesh=mesh,
      scratch_types=[
          pltpu.VMEM((CHUNK,), jnp.int32),
          pltpu.VMEM((CHUNK,), jnp.int32),
          pltpu.VMEM((CHUNK, DIM), jnp.float32),
          pltpu.VMEM((CHUNK, DIM), jnp.float32),
          pltpu.SemaphoreType.DMA,
          pltpu.SemaphoreType.DMA,
          pltpu.SemaphoreType.DMA,
          pltpu.SemaphoreType.DMA,
          pltpu.SemaphoreType.DMA,
          pltpu.SemaphoreType.DMA,
      ],
      compiler_params=pltpu.CompilerParams(use_tc_tiling_on_sc=False),
  )
  return f(x_flat, table)


def kernel(x, table):
  x_flat = x.reshape(-1).astype(jnp.int32)
  out = _emb(x_flat, table)
  return out.reshape(BATCH, HIST, DIM)
